# Initial kernel scaffold; baseline (speedup 1.0000x reference)
import jax
import jax.numpy as jnp
from jax.experimental import pallas as pl


def kernel(x, edge_index, global_features, batch, W_in, b_in, conv_W, conv_b, bn_g, bn_b, W_gp, b_gp, Wv1, bv1, Wv2, bv2, Wv3, bv3, hs_W1, hs_b1, hs_W2, hs_b2, hd_W1, hd_b1, hd_W2, hd_b2, ht_W1, ht_b1, ht_W2, ht_b2):
    raise NotImplementedError("write your pallas kernel here")


# speedup vs baseline: 5.9019x; 5.9019x over previous
import functools

import jax
import jax.numpy as jnp
import numpy as np
from jax import lax
from jax.experimental import pallas as pl
from jax.experimental.pallas import tpu as pltpu
from jax.experimental.pallas import tpu_sc as plsc

_N = 10000
_H = 128
_E = 320000
_NC = 2
_NS = 16
_RPT = 160
_CW = 64
_GR = 16
_EPAD = _NC * _NS * _RPT * _CW
_NPAD = 10112
_ZPT = _NPAD // _NS
_K = 3
_BNS = np.float32(np.sqrt(1.0 + 1e-5))

_sc_mesh = plsc.VectorSubcoreMesh(core_axis_name="c", subcore_axis_name="s")


def _sc_scatter_body(table, srcp, dstp, zeros, out,
                     si, di, b0, b1, b2, acc, sem):
    c = lax.axis_index("c")
    s = lax.axis_index("s")
    pltpu.sync_copy(zeros, b0)
    zb = s * _ZPT
    for k in range(10):
        sz = _CW if k < 9 else _ZPT - 9 * _CW
        pltpu.sync_copy(b0.at[pl.ds(0, sz)], acc.at[pl.ds(zb + k * _CW, sz)])
    plsc.subcore_barrier()

    bufs = (b0, b1, b2)

    def group(g, carry):
        r0 = pl.multiple_of(g * _GR, _GR)
        pltpu.sync_copy(srcp.at[c, s, pl.ds(r0, _GR)], si)
        pltpu.sync_copy(dstp.at[c, s, pl.ds(r0, _GR)], di)
        for r in range(_K):
            pltpu.async_copy(table.at[si.at[r]], bufs[r], sem)
        for r in range(_GR):
            pltpu.make_async_copy(table.at[si.at[r]], bufs[r % _K], sem).wait()
            pltpu.sync_copy(bufs[r % _K], acc.at[di.at[r]], add=True)
            if r + _K < _GR:
                pltpu.async_copy(
                    table.at[si.at[r + _K]], bufs[(r + _K) % _K], sem)
        return carry

    lax.fori_loop(0, _RPT // _GR, group, 0)
    plsc.subcore_barrier()
    ob = s * _ZPT
    for k in range(10):
        sz = _CW if k < 9 else _ZPT - 9 * _CW
        pltpu.sync_copy(acc.at[pl.ds(ob + k * _CW, sz)], b0.at[pl.ds(0, sz)])
        pltpu.sync_copy(b0.at[pl.ds(0, sz)], out.at[c, pl.ds(ob + k * _CW, sz)])


_sc_scatter = pl.kernel(
    _sc_scatter_body,
    out_type=jax.ShapeDtypeStruct((_NC, _NPAD, _H), jnp.float32),
    mesh=_sc_mesh,
    scratch_types=[
        pltpu.VMEM((_GR, _CW), jnp.int32),
        pltpu.VMEM((_GR, _CW), jnp.int32),
        pltpu.VMEM((_CW, _H), jnp.float32),
        pltpu.VMEM((_CW, _H), jnp.float32),
        pltpu.VMEM((_CW, _H), jnp.float32),
        pltpu.VMEM_SHARED((_NPAD, _H), jnp.float32),
        pltpu.SemaphoreType.DMA,
    ],
)


def _dot(a, b):
    return jnp.dot(a, b, preferred_element_type=jnp.float32)


def _tc_pre_body(x, w_in, bi, w0, degp, dinv_o, hw2_o):
    d = degp[...]
    deg = 1.0 + d[0, :_N, 0:1] + d[1, :_N, 0:1]
    dinv = lax.rsqrt(deg)
    h = jnp.maximum(_dot(x[...], w_in[...]) + bi[...], 0.0)
    hw2_o[...] = dinv * _dot(h, w0[...])
    dinv_o[...] = dinv


_tc_pre = pl.pallas_call(
    _tc_pre_body,
    out_shape=[
        jax.ShapeDtypeStruct((_N, 1), jnp.float32),
        jax.ShapeDtypeStruct((_N, _H), jnp.float32),
    ],
)


def _make_post(i):
    last = i == 2

    def body(*refs):
        if i == 0:
            accp, hw2, dinv, cb, g, b, wn, h_o, hw2_o = refs
            res = None
        elif i == 1:
            accp, hw2, dinv, cb, g, b, res, wn, h_o, hw2_o = refs
        else:
            accp, hw2, dinv, cb, g, b, res, h_o = refs
        a = accp[...]
        z = dinv[...] * (a[0, :_N] + a[1, :_N] + hw2[...]) + cb[...]
        z = z / _BNS * g[...] + b[...]
        h = jnp.maximum(z, 0.0)
        if res is not None:
            h = h + res[...]
        h_o[...] = h
        if not last:
            hw2_o[...] = dinv[...] * _dot(h, wn[...])

    outs = [jax.ShapeDtypeStruct((_N, _H), jnp.float32)]
    if not last:
        outs.append(jax.ShapeDtypeStruct((_N, _H), jnp.float32))
    return pl.pallas_call(body, out_shape=outs)


_tc_post = [_make_post(0), _make_post(1), _make_post(2)]


def _tc_pool_body(emb, brow, gf, wgp, bgp, wv1, bv1, wv2, bv2, wv3, bv3,
                  htw1, htb1, htw2, htb2, gemb_o, val_o, trs_o):
    rows = lax.broadcasted_iota(jnp.int32, (16, _N), 0)
    oh_t = (rows == brow[...]).astype(jnp.float32)
    sums = _dot(oh_t, emb[...])
    counts = jnp.sum(oh_t, axis=1, keepdims=True)
    graph_emb = sums / jnp.maximum(counts, 1.0)
    gemb = jnp.maximum(_dot(gf[...], wgp[...]) + bgp[...], 0.0)
    gemb_o[...] = gemb
    v = jnp.concatenate([graph_emb, gemb], axis=1)
    v = jnp.maximum(_dot(v, wv1[...]) + bv1[...], 0.0)
    v = jnp.maximum(_dot(v, wv2[...]) + bv2[...], 0.0)
    val_o[...] = _dot(v, wv3[...]) + bv3[...]
    w1 = htw1[...]
    b1 = htb1[...]
    w2 = htw2[...]
    b2 = htb2[...]
    for a in range(5):
        htr = jnp.maximum(_dot(gemb, w1[a]) + b1[a], 0.0)
        trs_o[a, :, :] = _dot(htr, w2[a]) + b2[a]


_tc_pool = pl.pallas_call(
    _tc_pool_body,
    out_shape=[
        jax.ShapeDtypeStruct((16, _H), jnp.float32),
        jax.ShapeDtypeStruct((16, 1), jnp.float32),
        jax.ShapeDtypeStruct((5, 16, 20), jnp.float32),
    ],
)

_HB = 1000


def _tc_heads_body(emb, bcol, gemb, ws1, bs1, ws2, bs2, wd1, bd1, wd2, bd2,
                   src_o, dst_o):
    cols = lax.broadcasted_iota(jnp.int32, (_HB, 16), 1)
    oh = (bcol[...] == cols).astype(jnp.float32)
    combined = jnp.concatenate([emb[...], _dot(oh, gemb[...])], axis=1)
    hs = jnp.maximum(_dot(combined, ws1[...]) + bs1[...], 0.0)
    src_o[...] = _dot(hs, ws2[...]) + bs2[...]
    hd = jnp.maximum(_dot(combined, wd1[...]) + bd1[...], 0.0)
    dst_o[...] = _dot(hd, wd2[...]) + bd2[...]


_tc_heads = pl.pallas_call(
    _tc_heads_body,
    grid=(_N // _HB,),
    in_specs=[
        pl.BlockSpec((_HB, _H), lambda i: (i, 0)),
        pl.BlockSpec((_HB, 1), lambda i: (i, 0)),
        pl.BlockSpec((16, _H), lambda i: (0, 0)),
        pl.BlockSpec((2 * _H, 5 * _H), lambda i: (0, 0)),
        pl.BlockSpec((1, 5 * _H), lambda i: (0, 0)),
        pl.BlockSpec((5 * _H, 5), lambda i: (0, 0)),
        pl.BlockSpec((1, 5), lambda i: (0, 0)),
        pl.BlockSpec((2 * _H, 5 * _H), lambda i: (0, 0)),
        pl.BlockSpec((1, 5 * _H), lambda i: (0, 0)),
        pl.BlockSpec((5 * _H, 5), lambda i: (0, 0)),
        pl.BlockSpec((1, 5), lambda i: (0, 0)),
    ],
    out_specs=[
        pl.BlockSpec((_HB, 5), lambda i: (i, 0)),
        pl.BlockSpec((_HB, 5), lambda i: (i, 0)),
    ],
    out_shape=[
        jax.ShapeDtypeStruct((_N, 5), jnp.float32),
        jax.ShapeDtypeStruct((_N, 5), jnp.float32),
    ],
)


def _blockdiag(w2):
    return (jnp.eye(5, dtype=w2.dtype)[:, None, :] * w2[:, :, None]).reshape(
        5 * _H, 5)


def kernel(x, edge_index, global_features, batch, W_in, b_in, conv_W, conv_b,
           bn_g, bn_b, W_gp, b_gp, Wv1, bv1, Wv2, bv2, Wv3, bv3,
           hs_W1, hs_b1, hs_W2, hs_b2, hd_W1, hd_b1, hd_W2, hd_b2,
           ht_W1, ht_b1, ht_W2, ht_b2):
    pad = _EPAD - _E
    srcp = jnp.concatenate(
        [edge_index[0], jnp.zeros((pad,), jnp.int32)]).reshape(
            _NC, _NS, _RPT, _CW)
    dstp = jnp.concatenate(
        [edge_index[1], jnp.full((pad,), _N, jnp.int32)]).reshape(
            _NC, _NS, _RPT, _CW)
    zeros = jnp.zeros((_CW, _H), jnp.float32)

    degp = _sc_scatter(jnp.ones((_N, _H), jnp.float32), srcp, dstp, zeros)
    dinv, hw2 = _tc_pre(x, W_in, b_in.reshape(1, _H), conv_W[0], degp)

    h = None
    node_emb = None
    for i in range(3):
        accp = _sc_scatter(hw2, srcp, dstp, zeros)
        cb = conv_b[i].reshape(1, _H)
        g = bn_g[i].reshape(1, _H)
        b = bn_b[i].reshape(1, _H)
        if i == 0:
            h, hw2 = _tc_post[0](accp, hw2, dinv, cb, g, b, conv_W[1])
        elif i == 1:
            h, hw2 = _tc_post[1](accp, hw2, dinv, cb, g, b, h, conv_W[2])
        else:
            (node_emb,) = _tc_post[2](accp, hw2, dinv, cb, g, b, h)

    g_emb, value, trs = _tc_pool(
        node_emb, batch.reshape(1, _N), global_features, W_gp,
        b_gp.reshape(1, _H), Wv1, bv1.reshape(1, _H), Wv2,
        bv2.reshape(1, _H // 2), Wv3, bv3.reshape(1, 1),
        ht_W1, ht_b1, ht_W2, ht_b2)

    ws1 = jnp.transpose(hs_W1, (1, 0, 2)).reshape(2 * _H, 5 * _H)
    wd1 = jnp.transpose(hd_W1, (1, 0, 2)).reshape(2 * _H, 5 * _H)
    srcs_nd, dsts_nd = _tc_heads(
        node_emb, batch.reshape(_N, 1), g_emb,
        ws1, hs_b1.reshape(1, 5 * _H), _blockdiag(hs_W2), hs_b2.reshape(1, 5),
        wd1, hd_b1.reshape(1, 5 * _H), _blockdiag(hd_W2), hd_b2.reshape(1, 5))

    return (jnp.transpose(srcs_nd), jnp.transpose(dsts_nd), trs, value)
